# baseline (speedup 1.0000x reference)
import functools

import jax
import jax.numpy as jnp
from jax import lax
from jax.experimental import pallas as pl
from jax.experimental.pallas import tpu as pltpu
from jax.experimental.pallas import tpu_sc as plsc

B = 8
N = 2048
M = 2048

N_SC = 256
NQ = 4
TCHUNK = N_SC // NQ
L = 16
G = 4
CU = 2

N_TC = N - N_SC
TN = 896
NT = N_TC // TN

BIG = 3.0e38


_GATHER_1D = lax.GatherDimensionNumbers(
    offset_dims=(), collapsed_slice_dims=(0,), start_index_map=(0,))


def _lane_shuffle(v, idx):
    return lax.gather(v, idx[:, None], _GATHER_1D, (1,),
                      mode=lax.GatherScatterMode.PROMISE_IN_BOUNDS)


def _lane_min(v, idx_consts):
    for sh in (8, 4, 2, 1):
        v = jnp.minimum(v, _lane_shuffle(v, idx_consts[sh]))
    return v


def _chamfer_sc_body(tmpl_hbm, src_hbm, minsrc_hbm, tsum_hbm,
                     sv, msv, tv, osv):
    wid = lax.axis_index("s") * 2 + lax.axis_index("c")
    _worker_body(wid, tmpl_hbm, src_hbm, minsrc_hbm, tsum_hbm,
                 sv, msv, tv, osv)


def _worker_body(wid, tmpl_hbm, src_hbm, minsrc_hbm, tsum_hbm,
                 sv, msv, tv, osv):
    b = wid // NQ
    q = wid % NQ

    pltpu.sync_copy(src_hbm.at[b], sv)
    pltpu.sync_copy(tmpl_hbm.at[b, q], tv)

    lanes = lax.iota(jnp.int32, L)
    splat_idx = [jnp.full((L,), j, dtype=jnp.int32) for j in range(L)]
    bfly_idx = {sh: lanes ^ sh for sh in (8, 4, 2, 1)}

    def init_ms(i, carry):
        msv[pl.ds(i * L, L)] = jnp.full((L,), BIG, dtype=jnp.float32)
        return carry

    lax.fori_loop(0, M // L, init_ms, 0)

    def tg_body(tg, ssum):
        tsl = pl.ds(tg * L, L)
        va0 = tv[0, tsl]
        va1 = tv[1, tsl]
        va2 = tv[2, tsl]
        vtn = tv[3, tsl]
        for sg in range(L // G):
            sp = []
            for k in range(G):
                idx = splat_idx[sg * G + k]
                sp.append((_lane_shuffle(va0, idx), _lane_shuffle(va1, idx),
                           _lane_shuffle(va2, idx), _lane_shuffle(vtn, idx)))

            def c_body(ci, accs):
                new = list(accs)
                for u in range(CU):
                    sl = pl.ds((ci * CU + u) * L, L)
                    sx = sv[0, sl]
                    sy = sv[1, sl]
                    sz = sv[2, sl]
                    sn = sv[3, sl]
                    mincur = msv[sl]
                    for k in range(G):
                        t0, t1, t2, tn = sp[k]
                        f = sx * t0 + sy * t1 + sz * t2 + sn
                        new[k] = jnp.minimum(new[k], f)
                        mincur = jnp.minimum(mincur, f + tn)
                    msv[sl] = mincur
                return tuple(new)

            inf = jnp.full((L,), BIG, dtype=jnp.float32)
            accs = lax.fori_loop(0, M // (L * CU), c_body, (inf,) * G)
            for k in range(G):
                ssum = ssum + (_lane_min(accs[k], bfly_idx) + sp[k][3])
        return ssum

    ssum = lax.fori_loop(0, TCHUNK // L, tg_body,
                         jnp.zeros((L,), dtype=jnp.float32))

    osv[...] = ssum
    pltpu.sync_copy(msv, minsrc_hbm.at[b, q])
    pltpu.sync_copy(osv, tsum_hbm.at[b, q])


@functools.lru_cache(maxsize=None)
def _build_chamfer_sc(interpret=False, num_cores=None, num_subcores=None):
    mesh_kwargs = {}
    if num_cores is not None:
        mesh_kwargs = dict(num_cores=num_cores, num_subcores=num_subcores)
    return functools.partial(
        pl.kernel,
        out_type=(
            jax.ShapeDtypeStruct((B, NQ, M), jnp.float32),
            jax.ShapeDtypeStruct((B, NQ, L), jnp.float32),
        ),
        mesh=plsc.VectorSubcoreMesh(core_axis_name="c", subcore_axis_name="s",
                                    **mesh_kwargs),
        scratch_types=[
            pltpu.VMEM((4, M), jnp.float32),
            pltpu.VMEM((M,), jnp.float32),
            pltpu.VMEM((4, TCHUNK), jnp.float32),
            pltpu.VMEM((L,), jnp.float32),
        ],
        interpret=interpret,
    )(_chamfer_sc_body)


def _chamfer_tc_body(tt_ref, src_ref, tn_ref, sn_ref, mint_ref, colmin_ref):
    tt = tt_ref[0]
    ss = src_ref[0]
    dot = lax.dot_general(tt, ss, (((1,), (1,)), ((), ())),
                          preferred_element_type=jnp.float32)
    d = (-2.0 * dot + tn_ref[0, 0, 0][:, None]) + sn_ref[0, 0][None, :]
    mint_ref[0, 0, 0] = jnp.min(d, axis=1)
    colmin_ref[0, 0, 0] = jnp.min(d, axis=0)


_chamfer_tc = pl.pallas_call(
    _chamfer_tc_body,
    grid=(B, NT),
    in_specs=[
        pl.BlockSpec((1, TN, 3), lambda b, i: (b, i, 0)),
        pl.BlockSpec((1, M, 3), lambda b, i: (b, 0, 0)),
        pl.BlockSpec((1, 1, 1, TN), lambda b, i: (b, i, 0, 0)),
        pl.BlockSpec((1, 1, M), lambda b, i: (b, 0, 0)),
    ],
    out_specs=(
        pl.BlockSpec((1, 1, 1, TN), lambda b, i: (b, i, 0, 0)),
        pl.BlockSpec((1, 1, 1, M), lambda b, i: (b, i, 0, 0)),
    ),
    out_shape=(
        jax.ShapeDtypeStruct((B, NT, 1, TN), jnp.float32),
        jax.ShapeDtypeStruct((B, NT, 1, M), jnp.float32),
    ),
)


def _finish_tc(colmin_ref, mint_ref, scmin_ref, sctsum_ref, out_ref):
    m = colmin_ref[:, 0, 0, :]
    for i in range(1, NT):
        m = jnp.minimum(m, colmin_ref[:, i, 0, :])
    for i in range(NQ):
        m = jnp.minimum(m, scmin_ref[:, i, :])
    total = (jnp.sum(m) + jnp.sum(mint_ref[...])
             + jnp.sum(sctsum_ref[...]) / L)
    out_ref[...] = jnp.full((1, 1), total / (N * B))


def _round_bf16(x):
    u = lax.bitcast_convert_type(x, jnp.uint32)
    u = u + jnp.uint32(0x7FFF) + ((u >> 16) & jnp.uint32(1))
    return lax.bitcast_convert_type(u & jnp.uint32(0xFFFF0000), jnp.float32)


def kernel(template, source):
    tmpl_r = _round_bf16(template)
    src_r = _round_bf16(source)
    tn = jnp.sum(template * template, -1)
    sn = jnp.sum(source * source, -1)

    tmpl_aug = jnp.concatenate(
        [tmpl_r[:, :N_SC] * -2.0, tn[:, :N_SC, None]], axis=-1)
    tmpl_soa = tmpl_aug.reshape(B, NQ, TCHUNK, 4).transpose(0, 1, 3, 2)
    src_aug = jnp.concatenate([src_r, sn[:, :, None]], axis=-1)
    src_aug = src_aug.transpose(0, 2, 1)

    tt16 = tmpl_r[:, N_SC:].astype(jnp.bfloat16)
    ss16 = src_r.astype(jnp.bfloat16)

    sc_minsrc, sc_tsum = _build_chamfer_sc()(tmpl_soa, src_aug)
    mint, colmin = _chamfer_tc(
        tt16, ss16,
        tn[:, N_SC:].reshape(B, NT, 1, TN),
        sn.reshape(B, 1, M),
    )
    out = pl.pallas_call(
        _finish_tc,
        out_shape=jax.ShapeDtypeStruct((1, 1), jnp.float32),
    )(colmin, mint, sc_minsrc, sc_tsum)
    return out[0, 0]


# speedup vs baseline: 1.0064x; 1.0064x over previous
import functools

import jax
import jax.numpy as jnp
from jax import lax
from jax.experimental import pallas as pl
from jax.experimental.pallas import tpu as pltpu
from jax.experimental.pallas import tpu_sc as plsc

B = 8
N = 2048
M = 2048

N_SC = 448
NQ = 4
TCHUNK = N_SC // NQ
L = 16
G = 4
CU = 2

N_TC = N - N_SC
TN = 400
NT = N_TC // TN

BIG = 3.0e38


_GATHER_1D = lax.GatherDimensionNumbers(
    offset_dims=(), collapsed_slice_dims=(0,), start_index_map=(0,))


def _lane_shuffle(v, idx):
    return lax.gather(v, idx[:, None], _GATHER_1D, (1,),
                      mode=lax.GatherScatterMode.PROMISE_IN_BOUNDS)


def _lane_min(v, idx_consts):
    for sh in (8, 4, 2, 1):
        v = jnp.minimum(v, _lane_shuffle(v, idx_consts[sh]))
    return v


def _chamfer_sc_body(tmpl_hbm, src_hbm, minsrc_hbm, tsum_hbm,
                     sv, msv, tv, osv):
    wid = lax.axis_index("s") * 2 + lax.axis_index("c")
    _worker_body(wid, tmpl_hbm, src_hbm, minsrc_hbm, tsum_hbm,
                 sv, msv, tv, osv)


def _worker_body(wid, tmpl_hbm, src_hbm, minsrc_hbm, tsum_hbm,
                 sv, msv, tv, osv):
    b = wid // NQ
    q = wid % NQ

    pltpu.sync_copy(src_hbm.at[b], sv)
    pltpu.sync_copy(tmpl_hbm.at[b, q], tv)

    lanes = lax.iota(jnp.int32, L)
    splat_idx = [jnp.full((L,), j, dtype=jnp.int32) for j in range(L)]
    bfly_idx = {sh: lanes ^ sh for sh in (8, 4, 2, 1)}

    def init_ms(i, carry):
        msv[pl.ds(i * L, L)] = jnp.full((L,), BIG, dtype=jnp.float32)
        return carry

    lax.fori_loop(0, M // L, init_ms, 0)

    def tg_body(tg, ssum):
        tsl = pl.ds(tg * L, L)
        va0 = tv[0, tsl]
        va1 = tv[1, tsl]
        va2 = tv[2, tsl]
        vtn = tv[3, tsl]
        for sg in range(L // G):
            sp = []
            for k in range(G):
                idx = splat_idx[sg * G + k]
                sp.append((_lane_shuffle(va0, idx), _lane_shuffle(va1, idx),
                           _lane_shuffle(va2, idx), _lane_shuffle(vtn, idx)))

            def c_body(ci, accs):
                new = list(accs)
                for u in range(CU):
                    sl = pl.ds((ci * CU + u) * L, L)
                    sx = sv[0, sl]
                    sy = sv[1, sl]
                    sz = sv[2, sl]
                    sn = sv[3, sl]
                    mincur = msv[sl]
                    for k in range(G):
                        t0, t1, t2, tn = sp[k]
                        f = sx * t0 + sy * t1 + sz * t2 + sn
                        new[k] = jnp.minimum(new[k], f)
                        mincur = jnp.minimum(mincur, f + tn)
                    msv[sl] = mincur
                return tuple(new)

            inf = jnp.full((L,), BIG, dtype=jnp.float32)
            accs = lax.fori_loop(0, M // (L * CU), c_body, (inf,) * G)
            for k in range(G):
                ssum = ssum + (_lane_min(accs[k], bfly_idx) + sp[k][3])
        return ssum

    ssum = lax.fori_loop(0, TCHUNK // L, tg_body,
                         jnp.zeros((L,), dtype=jnp.float32))

    osv[...] = ssum
    pltpu.sync_copy(msv, minsrc_hbm.at[b, q])
    pltpu.sync_copy(osv, tsum_hbm.at[b, q])


@functools.lru_cache(maxsize=None)
def _build_chamfer_sc(interpret=False, num_cores=None, num_subcores=None):
    mesh_kwargs = {}
    if num_cores is not None:
        mesh_kwargs = dict(num_cores=num_cores, num_subcores=num_subcores)
    return functools.partial(
        pl.kernel,
        out_type=(
            jax.ShapeDtypeStruct((B, NQ, M), jnp.float32),
            jax.ShapeDtypeStruct((B, NQ, L), jnp.float32),
        ),
        mesh=plsc.VectorSubcoreMesh(core_axis_name="c", subcore_axis_name="s",
                                    **mesh_kwargs),
        scratch_types=[
            pltpu.VMEM((4, M), jnp.float32),
            pltpu.VMEM((M,), jnp.float32),
            pltpu.VMEM((4, TCHUNK), jnp.float32),
            pltpu.VMEM((L,), jnp.float32),
        ],
        interpret=interpret,
    )(_chamfer_sc_body)


def _chamfer_tc_body(tt_ref, src_ref, tn_ref, sn_ref, mint_ref, colmin_ref):
    tt = tt_ref[0]
    ss = src_ref[0]
    dot = lax.dot_general(tt, ss, (((1,), (1,)), ((), ())),
                          preferred_element_type=jnp.float32)
    d = (-2.0 * dot + tn_ref[0, 0, 0][:, None]) + sn_ref[0, 0][None, :]
    mint_ref[0, 0, 0] = jnp.min(d, axis=1)
    colmin_ref[0, 0, 0] = jnp.min(d, axis=0)


_chamfer_tc = pl.pallas_call(
    _chamfer_tc_body,
    grid=(B, NT),
    in_specs=[
        pl.BlockSpec((1, TN, 3), lambda b, i: (b, i, 0)),
        pl.BlockSpec((1, M, 3), lambda b, i: (b, 0, 0)),
        pl.BlockSpec((1, 1, 1, TN), lambda b, i: (b, i, 0, 0)),
        pl.BlockSpec((1, 1, M), lambda b, i: (b, 0, 0)),
    ],
    out_specs=(
        pl.BlockSpec((1, 1, 1, TN), lambda b, i: (b, i, 0, 0)),
        pl.BlockSpec((1, 1, 1, M), lambda b, i: (b, i, 0, 0)),
    ),
    out_shape=(
        jax.ShapeDtypeStruct((B, NT, 1, TN), jnp.float32),
        jax.ShapeDtypeStruct((B, NT, 1, M), jnp.float32),
    ),
)


def _finish_tc(colmin_ref, mint_ref, scmin_ref, sctsum_ref, out_ref):
    m = colmin_ref[:, 0, 0, :]
    for i in range(1, NT):
        m = jnp.minimum(m, colmin_ref[:, i, 0, :])
    for i in range(NQ):
        m = jnp.minimum(m, scmin_ref[:, i, :])
    total = (jnp.sum(m) + jnp.sum(mint_ref[...])
             + jnp.sum(sctsum_ref[...]) / L)
    out_ref[...] = jnp.full((1, 1), total / (N * B))


def _round_bf16(x):
    u = lax.bitcast_convert_type(x, jnp.uint32)
    u = u + jnp.uint32(0x7FFF) + ((u >> 16) & jnp.uint32(1))
    return lax.bitcast_convert_type(u & jnp.uint32(0xFFFF0000), jnp.float32)


def kernel(template, source):
    tmpl_r = _round_bf16(template)
    src_r = _round_bf16(source)
    tn = jnp.sum(template * template, -1)
    sn = jnp.sum(source * source, -1)

    tmpl_aug = jnp.concatenate(
        [tmpl_r[:, :N_SC] * -2.0, tn[:, :N_SC, None]], axis=-1)
    tmpl_soa = tmpl_aug.reshape(B, NQ, TCHUNK, 4).transpose(0, 1, 3, 2)
    src_aug = jnp.concatenate([src_r, sn[:, :, None]], axis=-1)
    src_aug = src_aug.transpose(0, 2, 1)

    tt16 = tmpl_r[:, N_SC:].astype(jnp.bfloat16)
    ss16 = src_r.astype(jnp.bfloat16)

    sc_minsrc, sc_tsum = _build_chamfer_sc()(tmpl_soa, src_aug)
    mint, colmin = _chamfer_tc(
        tt16, ss16,
        tn[:, N_SC:].reshape(B, NT, 1, TN),
        sn.reshape(B, 1, M),
    )
    out = pl.pallas_call(
        _finish_tc,
        out_shape=jax.ShapeDtypeStruct((1, 1), jnp.float32),
    )(colmin, mint, sc_minsrc, sc_tsum)
    return out[0, 0]


# speedup vs baseline: 1.0594x; 1.0527x over previous
import functools

import jax
import jax.numpy as jnp
from jax import lax
from jax.experimental import pallas as pl
from jax.experimental.pallas import tpu as pltpu
from jax.experimental.pallas import tpu_sc as plsc

B = 8
N = 2048
M = 2048

N_SC = 384
NQ = 4
TCHUNK = N_SC // NQ
L = 16
G = 4
CU = 2

N_TC = N - N_SC
TN = 416
NT = N_TC // TN

BIG = 3.0e38


_GATHER_1D = lax.GatherDimensionNumbers(
    offset_dims=(), collapsed_slice_dims=(0,), start_index_map=(0,))


def _lane_shuffle(v, idx):
    return lax.gather(v, idx[:, None], _GATHER_1D, (1,),
                      mode=lax.GatherScatterMode.PROMISE_IN_BOUNDS)


def _lane_min(v, idx_consts):
    for sh in (8, 4, 2, 1):
        v = jnp.minimum(v, _lane_shuffle(v, idx_consts[sh]))
    return v


def _chamfer_sc_body(tmpl_hbm, src_hbm, minsrc_hbm, tsum_hbm,
                     sv, msv, tv, osv):
    wid = lax.axis_index("s") * 2 + lax.axis_index("c")
    _worker_body(wid, tmpl_hbm, src_hbm, minsrc_hbm, tsum_hbm,
                 sv, msv, tv, osv)


def _worker_body(wid, tmpl_hbm, src_hbm, minsrc_hbm, tsum_hbm,
                 sv, msv, tv, osv):
    b = wid // NQ
    q = wid % NQ

    pltpu.sync_copy(src_hbm.at[b], sv)
    pltpu.sync_copy(tmpl_hbm.at[b, q], tv)

    lanes = lax.iota(jnp.int32, L)
    splat_idx = [jnp.full((L,), j, dtype=jnp.int32) for j in range(L)]
    bfly_idx = {sh: lanes ^ sh for sh in (8, 4, 2, 1)}

    def init_ms(i, carry):
        msv[pl.ds(i * L, L)] = jnp.full((L,), BIG, dtype=jnp.float32)
        return carry

    lax.fori_loop(0, M // L, init_ms, 0)

    def tg_body(tg, ssum):
        tsl = pl.ds(tg * L, L)
        va0 = tv[0, tsl]
        va1 = tv[1, tsl]
        va2 = tv[2, tsl]
        vtn = tv[3, tsl]
        for sg in range(L // G):
            sp = []
            for k in range(G):
                idx = splat_idx[sg * G + k]
                sp.append((_lane_shuffle(va0, idx), _lane_shuffle(va1, idx),
                           _lane_shuffle(va2, idx), _lane_shuffle(vtn, idx)))

            def c_body(ci, accs):
                new = list(accs)
                for u in range(CU):
                    sl = pl.ds((ci * CU + u) * L, L)
                    sx = sv[0, sl]
                    sy = sv[1, sl]
                    sz = sv[2, sl]
                    sn = sv[3, sl]
                    mincur = msv[sl]
                    for k in range(G):
                        t0, t1, t2, tn = sp[k]
                        f = sx * t0 + sy * t1 + sz * t2 + sn
                        new[k] = jnp.minimum(new[k], f)
                        mincur = jnp.minimum(mincur, f + tn)
                    msv[sl] = mincur
                return tuple(new)

            inf = jnp.full((L,), BIG, dtype=jnp.float32)
            accs = lax.fori_loop(0, M // (L * CU), c_body, (inf,) * G)
            for k in range(G):
                ssum = ssum + (_lane_min(accs[k], bfly_idx) + sp[k][3])
        return ssum

    ssum = lax.fori_loop(0, TCHUNK // L, tg_body,
                         jnp.zeros((L,), dtype=jnp.float32))

    osv[...] = ssum
    pltpu.sync_copy(msv, minsrc_hbm.at[b, q])
    pltpu.sync_copy(osv, tsum_hbm.at[b, q])


@functools.lru_cache(maxsize=None)
def _build_chamfer_sc(interpret=False, num_cores=None, num_subcores=None):
    mesh_kwargs = {}
    if num_cores is not None:
        mesh_kwargs = dict(num_cores=num_cores, num_subcores=num_subcores)
    return functools.partial(
        pl.kernel,
        out_type=(
            jax.ShapeDtypeStruct((B, NQ, M), jnp.float32),
            jax.ShapeDtypeStruct((B, NQ, L), jnp.float32),
        ),
        mesh=plsc.VectorSubcoreMesh(core_axis_name="c", subcore_axis_name="s",
                                    **mesh_kwargs),
        scratch_types=[
            pltpu.VMEM((4, M), jnp.float32),
            pltpu.VMEM((M,), jnp.float32),
            pltpu.VMEM((4, TCHUNK), jnp.float32),
            pltpu.VMEM((L,), jnp.float32),
        ],
        interpret=interpret,
    )(_chamfer_sc_body)


def _chamfer_tc_body(tt_ref, src_ref, tn_ref, sn_ref, mint_ref, colmin_ref):
    tt = tt_ref[0]
    ss = src_ref[0]
    dot = lax.dot_general(tt, ss, (((1,), (1,)), ((), ())),
                          preferred_element_type=jnp.float32)
    d = (-2.0 * dot + tn_ref[0, 0, 0][:, None]) + sn_ref[0, 0][None, :]
    mint_ref[0, 0, 0] = jnp.min(d, axis=1)
    colmin_ref[0, 0, 0] = jnp.min(d, axis=0)


_chamfer_tc = pl.pallas_call(
    _chamfer_tc_body,
    grid=(B, NT),
    in_specs=[
        pl.BlockSpec((1, TN, 3), lambda b, i: (b, i, 0)),
        pl.BlockSpec((1, M, 3), lambda b, i: (b, 0, 0)),
        pl.BlockSpec((1, 1, 1, TN), lambda b, i: (b, i, 0, 0)),
        pl.BlockSpec((1, 1, M), lambda b, i: (b, 0, 0)),
    ],
    out_specs=(
        pl.BlockSpec((1, 1, 1, TN), lambda b, i: (b, i, 0, 0)),
        pl.BlockSpec((1, 1, 1, M), lambda b, i: (b, i, 0, 0)),
    ),
    out_shape=(
        jax.ShapeDtypeStruct((B, NT, 1, TN), jnp.float32),
        jax.ShapeDtypeStruct((B, NT, 1, M), jnp.float32),
    ),
)


def _finish_tc(colmin_ref, mint_ref, scmin_ref, sctsum_ref, out_ref):
    m = colmin_ref[:, 0, 0, :]
    for i in range(1, NT):
        m = jnp.minimum(m, colmin_ref[:, i, 0, :])
    for i in range(NQ):
        m = jnp.minimum(m, scmin_ref[:, i, :])
    total = (jnp.sum(m) + jnp.sum(mint_ref[...])
             + jnp.sum(sctsum_ref[...]) / L)
    out_ref[...] = jnp.full((1, 1), total / (N * B))


def _round_bf16(x):
    u = lax.bitcast_convert_type(x, jnp.uint32)
    u = u + jnp.uint32(0x7FFF) + ((u >> 16) & jnp.uint32(1))
    return lax.bitcast_convert_type(u & jnp.uint32(0xFFFF0000), jnp.float32)


def kernel(template, source):
    tmpl_r = _round_bf16(template)
    src_r = _round_bf16(source)
    tn = jnp.sum(template * template, -1)
    sn = jnp.sum(source * source, -1)

    tmpl_aug = jnp.concatenate(
        [tmpl_r[:, :N_SC] * -2.0, tn[:, :N_SC, None]], axis=-1)
    tmpl_soa = tmpl_aug.reshape(B, NQ, TCHUNK, 4).transpose(0, 1, 3, 2)
    src_aug = jnp.concatenate([src_r, sn[:, :, None]], axis=-1)
    src_aug = src_aug.transpose(0, 2, 1)

    tt16 = tmpl_r[:, N_SC:].astype(jnp.bfloat16)
    ss16 = src_r.astype(jnp.bfloat16)

    sc_minsrc, sc_tsum = _build_chamfer_sc()(tmpl_soa, src_aug)
    mint, colmin = _chamfer_tc(
        tt16, ss16,
        tn[:, N_SC:].reshape(B, NT, 1, TN),
        sn.reshape(B, 1, M),
    )
    out = pl.pallas_call(
        _finish_tc,
        out_shape=jax.ShapeDtypeStruct((1, 1), jnp.float32),
    )(colmin, mint, sc_minsrc, sc_tsum)
    return out[0, 0]
